# Initial kernel scaffold; baseline (speedup 1.0000x reference)
import jax
import jax.numpy as jnp
from jax.experimental import pallas as pl


def kernel(x, edge_index, edge_attr, batch, We1, be1, W11, b11, W12, b12, We2, be2, W21, b21, W22, b22, g1, bb1, Wf1, bf1, g2, bb2, Wf2, bf2):
    raise NotImplementedError("write your pallas kernel here")


# speedup vs baseline: 1.7559x; 1.7559x over previous
import functools

import jax
import jax.numpy as jnp
from jax import lax
from jax.experimental import pallas as pl
from jax.experimental.pallas import tpu as pltpu
from jax.experimental.pallas import tpu_sc as plsc

N = 10000
E = 160000
D = 256
DE = 16
H = 16
G = 64
C = 10

NT = 16
K2_CHUNK = 80
K4_CHUNK = 40
ROWS_PER_TILE = 624


def _k1_body(ea_ref, we1_ref, be1_ref, we2_ref, be2_ref, e1t_ref, e2_ref):
    ea = ea_ref[...]
    e1t_ref[...] = jnp.dot(ea, we1_ref[...],
                           preferred_element_type=jnp.float32) + be1_ref[0]
    e2_ref[...] = jnp.dot(ea, we2_ref[...],
                          preferred_element_type=jnp.float32) + be2_ref[...]


def _edge_mlps(edge_attr, We1, be1, We2, be2):
    BE = 4000
    nb = E // BE
    return pl.pallas_call(
        _k1_body,
        grid=(2, nb),
        in_specs=[
            pl.BlockSpec((BE, DE), lambda h, i: (i, 0)),
            pl.BlockSpec((DE, 128), lambda h, i: (0, h)),
            pl.BlockSpec((1, 1, 128), lambda h, i: (h, 0, 0)),
            pl.BlockSpec((DE, H), lambda h, i: (0, 0)),
            pl.BlockSpec((1, H), lambda h, i: (0, 0)),
        ],
        out_specs=[
            pl.BlockSpec((BE, 128), lambda h, i: (h * nb + i, 0)),
            pl.BlockSpec((BE, H), lambda h, i: (i, 0)),
        ],
        out_shape=[
            jax.ShapeDtypeStruct((2 * E, 128), jnp.float32),
            jax.ShapeDtypeStruct((E, H), jnp.float32),
        ],
    )(edge_attr, We1, be1.reshape(2, 1, 128), We2, be2.reshape(1, H))


def _k2_body(xcat, e1t, src, dst, out, acc, sidx, didx, gv, mv, sem):
    c = lax.axis_index("c")
    s = lax.axis_index("s")

    def _zrow(r, _):
        for j in range(128 // 16):
            mv[r, pl.ds(j * 16, 16)] = jnp.zeros((16,), jnp.float32)
        return 0

    lax.fori_loop(0, K2_CHUNK, _zrow, 0)
    row0 = s * ROWS_PER_TILE
    for k in range(7):
        pltpu.sync_copy(mv, acc.at[pl.ds(row0 + k * K2_CHUNK, K2_CHUNK)])
    pltpu.sync_copy(mv.at[pl.ds(0, 64)], acc.at[pl.ds(row0 + 560, 64)])

    @pl.when(s == NT - 1)
    def _():
        pltpu.sync_copy(mv.at[pl.ds(0, 16)], acc.at[pl.ds(N - 16, 16)])

    plsc.subcore_barrier()

    per_tile = E // NT

    def _chunk(i, _):
        base = s * per_tile + i * K2_CHUNK
        pltpu.sync_copy(src.at[pl.ds(base, K2_CHUNK)], sidx)
        pltpu.sync_copy(dst.at[pl.ds(base, K2_CHUNK)], didx)
        off = c * N
        for j in range(K2_CHUNK // 16):
            sl = pl.ds(j * 16, 16)
            sidx[sl] = sidx[sl] + off
        pltpu.sync_copy(e1t.at[pl.ds(c * E + base, K2_CHUNK)], mv)
        pltpu.async_copy(xcat.at[sidx], gv, sem).wait()

        def _row(r, _):
            for j in range(128 // 16):
                sl = pl.ds(j * 16, 16)
                mv[r, sl] = jnp.maximum(gv[r, sl] + mv[r, sl], 0.0)
            return 0

        lax.fori_loop(0, K2_CHUNK, _row, 0)
        pltpu.sync_copy(mv, acc.at[didx], add=True)
        return 0

    lax.fori_loop(0, per_tile // K2_CHUNK, _chunk, 0)
    plsc.subcore_barrier()
    pltpu.sync_copy(acc.at[pl.ds(row0, ROWS_PER_TILE)],
                    out.at[pl.ds(c * N + row0, ROWS_PER_TILE)])

    @pl.when(s == NT - 1)
    def _():
        pltpu.sync_copy(acc.at[pl.ds(N - 16, 16)],
                        out.at[pl.ds(c * N + N - 16, 16)])


def _layer1_aggregate(xcat, e1t, src, dst):
    mesh = plsc.VectorSubcoreMesh(core_axis_name="c", subcore_axis_name="s")
    return pl.kernel(
        _k2_body,
        mesh=mesh,
        out_type=jax.ShapeDtypeStruct((2 * N, 128), jnp.float32),
        scratch_types=[
            pltpu.VMEM_SHARED((N, 128), jnp.float32),
            pltpu.VMEM((K2_CHUNK,), jnp.int32),
            pltpu.VMEM((K2_CHUNK,), jnp.int32),
            pltpu.VMEM((K2_CHUNK, 128), jnp.float32),
            pltpu.VMEM((K2_CHUNK, 128), jnp.float32),
            pltpu.SemaphoreType.DMA,
        ],
    )(xcat, e1t, src, dst)


def _k3_body(x_ref, a0_ref, a1_ref, w11_ref, b11_ref, w12_ref, b12_ref,
             g1_ref, bb1_ref, out_ref):
    h = x_ref[...] + jnp.concatenate([a0_ref[...], a1_ref[...]], axis=-1)
    t = jnp.maximum(jnp.dot(h, w11_ref[...],
                            preferred_element_type=jnp.float32) + b11_ref[...], 0.0)
    t = jnp.dot(t, w12_ref[...], preferred_element_type=jnp.float32) + b12_ref[...]
    t = jnp.maximum(t, 0.0)
    mu = jnp.mean(t, axis=-1, keepdims=True)
    var = jnp.mean((t - mu) ** 2, axis=-1, keepdims=True)
    t = (t - mu) / jnp.sqrt(var + 1e-5) * g1_ref[...] + bb1_ref[...]
    out_ref[...] = jnp.concatenate(
        [t, jnp.zeros((t.shape[0], 128 - H), jnp.float32)], axis=-1)


def _node_mlp1(x, a0, a1, W11, b11, W12, b12, g1, bb1):
    BN = 1000
    nb = N // BN
    return pl.pallas_call(
        _k3_body,
        grid=(nb,),
        in_specs=[
            pl.BlockSpec((BN, D), lambda i: (i, 0)),
            pl.BlockSpec((BN, 128), lambda i: (i, 0)),
            pl.BlockSpec((BN, 128), lambda i: (i, 0)),
            pl.BlockSpec((D, H), lambda i: (0, 0)),
            pl.BlockSpec((1, H), lambda i: (0, 0)),
            pl.BlockSpec((H, H), lambda i: (0, 0)),
            pl.BlockSpec((1, H), lambda i: (0, 0)),
            pl.BlockSpec((1, H), lambda i: (0, 0)),
            pl.BlockSpec((1, H), lambda i: (0, 0)),
        ],
        out_specs=pl.BlockSpec((BN, 128), lambda i: (i, 0)),
        out_shape=jax.ShapeDtypeStruct((N, 128), jnp.float32),
    )(x, a0, a1, W11, b11.reshape(1, H), W12, b12.reshape(1, H),
      g1.reshape(1, H), bb1.reshape(1, H))


def _k4_body(h1, e2, src, dst, out, acc, sidx, didx, gv, mv, ev, sem):
    c = lax.axis_index("c")
    s = lax.axis_index("s")
    w = s * 2 + c

    def _zrow(r, _):
        for j in range(128 // 16):
            mv[r, pl.ds(j * 16, 16)] = jnp.zeros((16,), jnp.float32)
        return 0

    lax.fori_loop(0, K4_CHUNK, _zrow, 0)
    row0 = s * ROWS_PER_TILE
    for k in range(15):
        pltpu.sync_copy(mv, acc.at[pl.ds(row0 + k * K4_CHUNK, K4_CHUNK)])
    pltpu.sync_copy(mv.at[pl.ds(0, 24)], acc.at[pl.ds(row0 + 600, 24)])

    @pl.when(s == NT - 1)
    def _():
        pltpu.sync_copy(mv.at[pl.ds(0, 16)], acc.at[pl.ds(N - 16, 16)])

    plsc.subcore_barrier()

    per_w = E // 32

    def _chunk(i, _):
        base = w * per_w + i * K4_CHUNK
        pltpu.sync_copy(src.at[pl.ds(base, K4_CHUNK)], sidx)
        pltpu.sync_copy(dst.at[pl.ds(base, K4_CHUNK)], didx)
        pltpu.sync_copy(e2.at[pl.ds(base, K4_CHUNK)], ev)
        pltpu.async_copy(h1.at[sidx], gv, sem).wait()

        def _row(r, _):
            mv[r, pl.ds(0, 16)] = jnp.maximum(
                gv[r, pl.ds(0, 16)] + ev[r, :], 0.0)
            return 0

        lax.fori_loop(0, K4_CHUNK, _row, 0)
        pltpu.sync_copy(mv, acc.at[didx], add=True)
        return 0

    lax.fori_loop(0, per_w // K4_CHUNK, _chunk, 0)
    plsc.subcore_barrier()
    pltpu.sync_copy(acc.at[pl.ds(row0, ROWS_PER_TILE)],
                    out.at[pl.ds(c * N + row0, ROWS_PER_TILE)])

    @pl.when(s == NT - 1)
    def _():
        pltpu.sync_copy(acc.at[pl.ds(N - 16, 16)],
                        out.at[pl.ds(c * N + N - 16, 16)])


def _layer2_aggregate(h1, e2, src, dst):
    mesh = plsc.VectorSubcoreMesh(core_axis_name="c", subcore_axis_name="s")
    return pl.kernel(
        _k4_body,
        mesh=mesh,
        out_type=jax.ShapeDtypeStruct((2 * N, 128), jnp.float32),
        scratch_types=[
            pltpu.VMEM_SHARED((N, 128), jnp.float32),
            pltpu.VMEM((K4_CHUNK,), jnp.int32),
            pltpu.VMEM((K4_CHUNK,), jnp.int32),
            pltpu.VMEM((K4_CHUNK, 128), jnp.float32),
            pltpu.VMEM((K4_CHUNK, 128), jnp.float32),
            pltpu.VMEM((K4_CHUNK, H), jnp.float32),
            pltpu.SemaphoreType.DMA,
        ],
    )(h1, e2, src, dst)


def _k5_body(h1_ref, p0_ref, p1_ref, b_ref, w21_ref, b21_ref, w22_ref,
             b22_ref, wf1_ref, bf1_ref, g2_ref, bb2_ref, wf2_ref, bf2_ref,
             out_ref, pooled):
    i = pl.program_id(0)
    nb = pl.num_programs(0)
    hh = (h1_ref[:, :H] + p0_ref[:, :H] + p1_ref[:, :H])
    t = jnp.maximum(jnp.dot(hh, w21_ref[...],
                            preferred_element_type=jnp.float32) + b21_ref[...], 0.0)
    t = jnp.dot(t, w22_ref[...], preferred_element_type=jnp.float32) + b22_ref[...]
    h2 = jnp.maximum(t, 0.0)
    bid = b_ref[...]
    mask = bid == lax.broadcasted_iota(jnp.int32, (1, G), 1)
    rows = []
    for f in range(H):
        wf = jnp.where(mask, h2[:, f:f + 1], -jnp.inf)
        rows.append(jnp.max(wf, axis=0, keepdims=True))
    bmax = jnp.concatenate(rows, axis=0)

    @pl.when(i == 0)
    def _():
        pooled[...] = jnp.full((H, G), -jnp.inf, jnp.float32)

    pooled[...] = jnp.maximum(pooled[...], bmax)

    @pl.when(i == nb - 1)
    def _():
        y = lax.dot_general(pooled[...], wf1_ref[...],
                            (((0,), (0,)), ((), ())),
                            preferred_element_type=jnp.float32) + bf1_ref[...]
        mu = jnp.mean(y, axis=-1, keepdims=True)
        var = jnp.mean((y - mu) ** 2, axis=-1, keepdims=True)
        y = (y - mu) / jnp.sqrt(var + 1e-5) * g2_ref[...] + bb2_ref[...]
        y = jnp.maximum(y, 0.0)
        y = jnp.dot(y, wf2_ref[...],
                    preferred_element_type=jnp.float32) + bf2_ref[...]
        m = jnp.max(y, axis=-1, keepdims=True)
        z = y - m
        out_ref[...] = z - jnp.log(jnp.sum(jnp.exp(z), axis=-1, keepdims=True))


def _pool_head(h1, p0, p1, batch, W21, b21, W22, b22, Wf1, bf1, g2, bb2,
               Wf2, bf2):
    BN = 1000
    nb = N // BN
    full = lambda i: (0, 0)
    return pl.pallas_call(
        _k5_body,
        grid=(nb,),
        in_specs=[
            pl.BlockSpec((BN, 128), lambda i: (i, 0)),
            pl.BlockSpec((BN, 128), lambda i: (i, 0)),
            pl.BlockSpec((BN, 128), lambda i: (i, 0)),
            pl.BlockSpec((BN, 1), lambda i: (i, 0)),
            pl.BlockSpec((H, H), full),
            pl.BlockSpec((1, H), full),
            pl.BlockSpec((H, H), full),
            pl.BlockSpec((1, H), full),
            pl.BlockSpec((H, 32), full),
            pl.BlockSpec((1, 32), full),
            pl.BlockSpec((1, 32), full),
            pl.BlockSpec((1, 32), full),
            pl.BlockSpec((32, C), full),
            pl.BlockSpec((1, C), full),
        ],
        out_specs=pl.BlockSpec((G, C), full),
        out_shape=jax.ShapeDtypeStruct((G, C), jnp.float32),
        scratch_shapes=[pltpu.VMEM((H, G), jnp.float32)],
    )(h1, p0, p1, batch.reshape(N, 1), W21, b21.reshape(1, H), W22,
      b22.reshape(1, H), Wf1, bf1.reshape(1, 32), g2.reshape(1, 32),
      bb2.reshape(1, 32), Wf2, bf2.reshape(1, C))


def kernel(x, edge_index, edge_attr, batch, We1, be1, W11, b11, W12, b12,
           We2, be2, W21, b21, W22, b22, g1, bb1, Wf1, bf1, g2, bb2, Wf2,
           bf2):
    src = edge_index[0]
    dst = edge_index[1]
    e1t, e2 = _edge_mlps(edge_attr, We1, be1, We2, be2)
    xcat = jnp.concatenate([x[:, :128], x[:, 128:]], axis=0)
    aggr = _layer1_aggregate(xcat, e1t, src, dst)
    h1 = _node_mlp1(x, aggr[:N], aggr[N:], W11, b11, W12, b12, g1, bb1)
    pp = _layer2_aggregate(h1, e2, src, dst)
    return _pool_head(h1, pp[:N], pp[N:], batch, W21, b21, W22, b22,
                      Wf1, bf1, g2, bb2, Wf2, bf2)


# speedup vs baseline: 2.7580x; 1.5707x over previous
import functools

import jax
import jax.numpy as jnp
from jax import lax
from jax.experimental import pallas as pl
from jax.experimental.pallas import tpu as pltpu
from jax.experimental.pallas import tpu_sc as plsc

N = 10000
E = 160000
D = 256
DE = 16
H = 16
G = 64
C = 10

NT = 16
K2_CHUNK = 80
K4_CHUNK = 40
ROWS_PER_TILE = 624


def _k1_body(ea_ref, we1_ref, be1_ref, we2_ref, be2_ref, e1t_ref, e2_ref):
    ea = ea_ref[...]
    e1t_ref[...] = jnp.dot(ea, we1_ref[...],
                           preferred_element_type=jnp.float32) + be1_ref[0]
    e2_ref[...] = jnp.dot(ea, we2_ref[...],
                          preferred_element_type=jnp.float32) + be2_ref[...]


def _edge_mlps(edge_attr, We1, be1, We2, be2):
    BE = 4000
    nb = E // BE
    return pl.pallas_call(
        _k1_body,
        grid=(2, nb),
        in_specs=[
            pl.BlockSpec((BE, DE), lambda h, i: (i, 0)),
            pl.BlockSpec((DE, 128), lambda h, i: (0, h)),
            pl.BlockSpec((1, 1, 128), lambda h, i: (h, 0, 0)),
            pl.BlockSpec((DE, H), lambda h, i: (0, 0)),
            pl.BlockSpec((1, H), lambda h, i: (0, 0)),
        ],
        out_specs=[
            pl.BlockSpec((BE, 128), lambda h, i: (h * nb + i, 0)),
            pl.BlockSpec((BE, H), lambda h, i: (i, 0)),
        ],
        out_shape=[
            jax.ShapeDtypeStruct((2 * E, 128), jnp.float32),
            jax.ShapeDtypeStruct((E, H), jnp.float32),
        ],
    )(edge_attr, We1, be1.reshape(2, 1, 128), We2, be2.reshape(1, H))


NB = 2


def _k2_body(xcat, e1t, srccat, dst, out, acc, sidx, didx, gv, mv,
             sem_in, sem_g):
    c = lax.axis_index("c")
    s = lax.axis_index("s")
    per_tile = E // NT
    n_chunks = per_tile // K2_CHUNK

    def _zrow(r, _):
        for j in range(128 // 16):
            mv[0, r, pl.ds(j * 16, 16)] = jnp.zeros((16,), jnp.float32)
        return 0

    lax.fori_loop(0, K2_CHUNK, _zrow, 0)
    row0 = s * ROWS_PER_TILE
    for k in range(7):
        pltpu.sync_copy(mv.at[0], acc.at[pl.ds(row0 + k * K2_CHUNK, K2_CHUNK)])
    pltpu.sync_copy(mv.at[0, pl.ds(0, 64)], acc.at[pl.ds(row0 + 560, 64)])

    @pl.when(s == NT - 1)
    def _():
        pltpu.sync_copy(mv.at[0, pl.ds(0, 16)], acc.at[pl.ds(N - 16, 16)])

    plsc.subcore_barrier()

    def _in_copies(i, b):
        base = s * per_tile + i * K2_CHUNK
        a = pltpu.async_copy(srccat.at[pl.ds(c * E + base, K2_CHUNK)],
                             sidx.at[b], sem_in.at[b])
        pltpu.async_copy(dst.at[pl.ds(base, K2_CHUNK)], didx.at[b],
                         sem_in.at[b])
        pltpu.async_copy(e1t.at[pl.ds(c * E + base, K2_CHUNK)], mv.at[b],
                         sem_in.at[b])
        del a

    def _wait_in(i, b):
        base = s * per_tile + i * K2_CHUNK
        pltpu.make_async_copy(srccat.at[pl.ds(c * E + base, K2_CHUNK)],
                              sidx.at[b], sem_in.at[b]).wait()
        pltpu.make_async_copy(dst.at[pl.ds(base, K2_CHUNK)], didx.at[b],
                              sem_in.at[b]).wait()
        pltpu.make_async_copy(e1t.at[pl.ds(c * E + base, K2_CHUNK)],
                              mv.at[b], sem_in.at[b]).wait()

    def _gather(b):
        pltpu.async_copy(xcat.at[sidx.at[b]], gv.at[b], sem_g.at[b])

    def _wait_gather(b):
        pltpu.make_async_copy(xcat.at[sidx.at[b]], gv.at[b],
                              sem_g.at[b]).wait()

    def _compute_scatter(b):
        def _row(r, _):
            for j in range(128 // 16):
                sl = pl.ds(j * 16, 16)
                mv[b, r, sl] = jnp.maximum(gv[b, r, sl] + mv[b, r, sl], 0.0)
            return 0

        lax.fori_loop(0, K2_CHUNK, _row, 0)
        pltpu.sync_copy(mv.at[b], acc.at[didx.at[b]], add=True)

    _in_copies(0, 0)
    _in_copies(1, 1)
    _wait_in(0, 0)
    _gather(0)

    def _step(i, b, nb, do_copies, last=False):
        if not last:
            _wait_in(i + 1, nb)
            _gather(nb)
        _wait_gather(b)
        _compute_scatter(b)
        if do_copies:
            _in_copies(i + 2, b)

    def _outer(o, _):
        i0 = o * 2
        for k in range(2):
            _step(i0 + k, k, 1 - k, True)
        return 0

    lax.fori_loop(0, (n_chunks - 3) // 2, _outer, 0)
    _step(n_chunks - 3, 0, 1, True)
    _step(n_chunks - 2, 1, 0, False)
    _step(n_chunks - 1, 0, None, False, last=True)

    plsc.subcore_barrier()
    pltpu.sync_copy(acc.at[pl.ds(row0, ROWS_PER_TILE)],
                    out.at[pl.ds(c * N + row0, ROWS_PER_TILE)])

    @pl.when(s == NT - 1)
    def _():
        pltpu.sync_copy(acc.at[pl.ds(N - 16, 16)],
                        out.at[pl.ds(c * N + N - 16, 16)])


def _layer1_aggregate(xcat, e1t, srccat, dst):
    mesh = plsc.VectorSubcoreMesh(core_axis_name="c", subcore_axis_name="s")
    return pl.kernel(
        _k2_body,
        mesh=mesh,
        out_type=jax.ShapeDtypeStruct((2 * N, 128), jnp.float32),
        scratch_types=[
            pltpu.VMEM_SHARED((N, 128), jnp.float32),
            pltpu.VMEM((NB, K2_CHUNK), jnp.int32),
            pltpu.VMEM((NB, K2_CHUNK), jnp.int32),
            pltpu.VMEM((NB, K2_CHUNK, 128), jnp.float32),
            pltpu.VMEM((NB, K2_CHUNK, 128), jnp.float32),
            pltpu.SemaphoreType.DMA((NB,)),
            pltpu.SemaphoreType.DMA((NB,)),
        ],
    )(xcat, e1t, srccat, dst)


def _k3_body(x_ref, a0_ref, a1_ref, w11_ref, b11_ref, w12_ref, b12_ref,
             g1_ref, bb1_ref, out_ref):
    h = x_ref[...] + jnp.concatenate([a0_ref[...], a1_ref[...]], axis=-1)
    t = jnp.maximum(jnp.dot(h, w11_ref[...],
                            preferred_element_type=jnp.float32) + b11_ref[...], 0.0)
    t = jnp.dot(t, w12_ref[...], preferred_element_type=jnp.float32) + b12_ref[...]
    t = jnp.maximum(t, 0.0)
    mu = jnp.mean(t, axis=-1, keepdims=True)
    var = jnp.mean((t - mu) ** 2, axis=-1, keepdims=True)
    t = (t - mu) / jnp.sqrt(var + 1e-5) * g1_ref[...] + bb1_ref[...]
    out_ref[...] = jnp.concatenate(
        [t, jnp.zeros((t.shape[0], 128 - H), jnp.float32)], axis=-1)


def _node_mlp1(x, a0, a1, W11, b11, W12, b12, g1, bb1):
    BN = 1000
    nb = N // BN
    return pl.pallas_call(
        _k3_body,
        grid=(nb,),
        in_specs=[
            pl.BlockSpec((BN, D), lambda i: (i, 0)),
            pl.BlockSpec((BN, 128), lambda i: (i, 0)),
            pl.BlockSpec((BN, 128), lambda i: (i, 0)),
            pl.BlockSpec((D, H), lambda i: (0, 0)),
            pl.BlockSpec((1, H), lambda i: (0, 0)),
            pl.BlockSpec((H, H), lambda i: (0, 0)),
            pl.BlockSpec((1, H), lambda i: (0, 0)),
            pl.BlockSpec((1, H), lambda i: (0, 0)),
            pl.BlockSpec((1, H), lambda i: (0, 0)),
        ],
        out_specs=pl.BlockSpec((BN, 128), lambda i: (i, 0)),
        out_shape=jax.ShapeDtypeStruct((N, 128), jnp.float32),
    )(x, a0, a1, W11, b11.reshape(1, H), W12, b12.reshape(1, H),
      g1.reshape(1, H), bb1.reshape(1, H))


def _k4_body(h1, e2, src, dst, out, acc, sidx, didx, gv, mv, ev,
             sem_in, sem_g):
    c = lax.axis_index("c")
    s = lax.axis_index("s")
    w = s * 2 + c
    per_w = E // 32
    n_chunks = per_w // K4_CHUNK

    def _zrow(r, _):
        for j in range(128 // 16):
            mv[0, r, pl.ds(j * 16, 16)] = jnp.zeros((16,), jnp.float32)
        return 0

    lax.fori_loop(0, K4_CHUNK, _zrow, 0)
    for b in range(1, NB):
        def _zrow_b(r, _, b=b):
            for j in range(128 // 16):
                mv[b, r, pl.ds(j * 16, 16)] = jnp.zeros((16,), jnp.float32)
            return 0

        lax.fori_loop(0, K4_CHUNK, _zrow_b, 0)
    row0 = s * ROWS_PER_TILE
    for k in range(15):
        pltpu.sync_copy(mv.at[0], acc.at[pl.ds(row0 + k * K4_CHUNK, K4_CHUNK)])
    pltpu.sync_copy(mv.at[0, pl.ds(0, 24)], acc.at[pl.ds(row0 + 600, 24)])

    @pl.when(s == NT - 1)
    def _():
        pltpu.sync_copy(mv.at[0, pl.ds(0, 16)], acc.at[pl.ds(N - 16, 16)])

    plsc.subcore_barrier()

    def _in_copies(i, b):
        base = w * per_w + i * K4_CHUNK
        pltpu.async_copy(src.at[pl.ds(base, K4_CHUNK)], sidx.at[b],
                         sem_in.at[b])
        pltpu.async_copy(dst.at[pl.ds(base, K4_CHUNK)], didx.at[b],
                         sem_in.at[b])
        pltpu.async_copy(e2.at[pl.ds(base, K4_CHUNK)], ev.at[b],
                         sem_in.at[b])

    def _wait_in(i, b):
        base = w * per_w + i * K4_CHUNK
        pltpu.make_async_copy(src.at[pl.ds(base, K4_CHUNK)], sidx.at[b],
                              sem_in.at[b]).wait()
        pltpu.make_async_copy(dst.at[pl.ds(base, K4_CHUNK)], didx.at[b],
                              sem_in.at[b]).wait()
        pltpu.make_async_copy(e2.at[pl.ds(base, K4_CHUNK)], ev.at[b],
                              sem_in.at[b]).wait()

    def _gather(b):
        pltpu.async_copy(h1.at[sidx.at[b]], gv.at[b], sem_g.at[b])

    def _wait_gather(b):
        pltpu.make_async_copy(h1.at[sidx.at[b]], gv.at[b],
                              sem_g.at[b]).wait()

    def _compute_scatter(b):
        def _row(r, _):
            mv[b, r, pl.ds(0, 16)] = jnp.maximum(
                gv[b, r, pl.ds(0, 16)] + ev[b, r, :], 0.0)
            return 0

        lax.fori_loop(0, K4_CHUNK, _row, 0)
        pltpu.sync_copy(mv.at[b], acc.at[didx.at[b]], add=True)

    _in_copies(0, 0)
    _in_copies(1, 1)
    _wait_in(0, 0)
    _gather(0)

    def _step(i, b, nb, do_copies, last=False):
        if not last:
            _wait_in(i + 1, nb)
            _gather(nb)
        _wait_gather(b)
        _compute_scatter(b)
        if do_copies:
            _in_copies(i + 2, b)

    def _outer(o, _):
        i0 = o * 2
        for k in range(2):
            _step(i0 + k, k, 1 - k, True)
        return 0

    lax.fori_loop(0, (n_chunks - 3) // 2, _outer, 0)
    _step(n_chunks - 3, 0, 1, True)
    _step(n_chunks - 2, 1, 0, False)
    _step(n_chunks - 1, 0, None, False, last=True)

    plsc.subcore_barrier()
    pltpu.sync_copy(acc.at[pl.ds(row0, ROWS_PER_TILE)],
                    out.at[pl.ds(c * N + row0, ROWS_PER_TILE)])

    @pl.when(s == NT - 1)
    def _():
        pltpu.sync_copy(acc.at[pl.ds(N - 16, 16)],
                        out.at[pl.ds(c * N + N - 16, 16)])


def _layer2_aggregate(h1, e2, src, dst):
    mesh = plsc.VectorSubcoreMesh(core_axis_name="c", subcore_axis_name="s")
    return pl.kernel(
        _k4_body,
        mesh=mesh,
        out_type=jax.ShapeDtypeStruct((2 * N, 128), jnp.float32),
        scratch_types=[
            pltpu.VMEM_SHARED((N, 128), jnp.float32),
            pltpu.VMEM((NB, K4_CHUNK), jnp.int32),
            pltpu.VMEM((NB, K4_CHUNK), jnp.int32),
            pltpu.VMEM((NB, K4_CHUNK, 128), jnp.float32),
            pltpu.VMEM((NB, K4_CHUNK, 128), jnp.float32),
            pltpu.VMEM((NB, K4_CHUNK, H), jnp.float32),
            pltpu.SemaphoreType.DMA((NB,)),
            pltpu.SemaphoreType.DMA((NB,)),
        ],
    )(h1, e2, src, dst)


def _k5_body(h1_ref, p0_ref, p1_ref, b_ref, w21_ref, b21_ref, w22_ref,
             b22_ref, wf1_ref, bf1_ref, g2_ref, bb2_ref, wf2_ref, bf2_ref,
             out_ref, pooled):
    i = pl.program_id(0)
    nb = pl.num_programs(0)
    hh = (h1_ref[:, :H] + p0_ref[:, :H] + p1_ref[:, :H])
    t = jnp.maximum(jnp.dot(hh, w21_ref[...],
                            preferred_element_type=jnp.float32) + b21_ref[...], 0.0)
    t = jnp.dot(t, w22_ref[...], preferred_element_type=jnp.float32) + b22_ref[...]
    h2 = jnp.maximum(t, 0.0)
    bid = b_ref[...]
    mask = bid == lax.broadcasted_iota(jnp.int32, (1, G), 1)
    rows = []
    for f in range(H):
        wf = jnp.where(mask, h2[:, f:f + 1], -jnp.inf)
        rows.append(jnp.max(wf, axis=0, keepdims=True))
    bmax = jnp.concatenate(rows, axis=0)

    @pl.when(i == 0)
    def _():
        pooled[...] = jnp.full((H, G), -jnp.inf, jnp.float32)

    pooled[...] = jnp.maximum(pooled[...], bmax)

    @pl.when(i == nb - 1)
    def _():
        y = lax.dot_general(pooled[...], wf1_ref[...],
                            (((0,), (0,)), ((), ())),
                            preferred_element_type=jnp.float32) + bf1_ref[...]
        mu = jnp.mean(y, axis=-1, keepdims=True)
        var = jnp.mean((y - mu) ** 2, axis=-1, keepdims=True)
        y = (y - mu) / jnp.sqrt(var + 1e-5) * g2_ref[...] + bb2_ref[...]
        y = jnp.maximum(y, 0.0)
        y = jnp.dot(y, wf2_ref[...],
                    preferred_element_type=jnp.float32) + bf2_ref[...]
        m = jnp.max(y, axis=-1, keepdims=True)
        z = y - m
        out_ref[...] = z - jnp.log(jnp.sum(jnp.exp(z), axis=-1, keepdims=True))


def _pool_head(h1, p0, p1, batch, W21, b21, W22, b22, Wf1, bf1, g2, bb2,
               Wf2, bf2):
    BN = 1000
    nb = N // BN
    full = lambda i: (0, 0)
    return pl.pallas_call(
        _k5_body,
        grid=(nb,),
        in_specs=[
            pl.BlockSpec((BN, 128), lambda i: (i, 0)),
            pl.BlockSpec((BN, 128), lambda i: (i, 0)),
            pl.BlockSpec((BN, 128), lambda i: (i, 0)),
            pl.BlockSpec((BN, 1), lambda i: (i, 0)),
            pl.BlockSpec((H, H), full),
            pl.BlockSpec((1, H), full),
            pl.BlockSpec((H, H), full),
            pl.BlockSpec((1, H), full),
            pl.BlockSpec((H, 32), full),
            pl.BlockSpec((1, 32), full),
            pl.BlockSpec((1, 32), full),
            pl.BlockSpec((1, 32), full),
            pl.BlockSpec((32, C), full),
            pl.BlockSpec((1, C), full),
        ],
        out_specs=pl.BlockSpec((G, C), full),
        out_shape=jax.ShapeDtypeStruct((G, C), jnp.float32),
        scratch_shapes=[pltpu.VMEM((H, G), jnp.float32)],
    )(h1, p0, p1, batch.reshape(N, 1), W21, b21.reshape(1, H), W22,
      b22.reshape(1, H), Wf1, bf1.reshape(1, 32), g2.reshape(1, 32),
      bb2.reshape(1, 32), Wf2, bf2.reshape(1, C))


def kernel(x, edge_index, edge_attr, batch, We1, be1, W11, b11, W12, b12,
           We2, be2, W21, b21, W22, b22, g1, bb1, Wf1, bf1, g2, bb2, Wf2,
           bf2):
    src = edge_index[0]
    dst = edge_index[1]
    srccat = jnp.concatenate([src, src + N])
    e1t, e2 = _edge_mlps(edge_attr, We1, be1, We2, be2)
    xcat = jnp.concatenate([x[:, :128], x[:, 128:]], axis=0)
    aggr = _layer1_aggregate(xcat, e1t, srccat, dst)
    h1 = _node_mlp1(x, aggr[:N], aggr[N:], W11, b11, W12, b12, g1, bb1)
    pp = _layer2_aggregate(h1, e2, src, dst)
    return _pool_head(h1, pp[:N], pp[N:], batch, W21, b21, W22, b22,
                      Wf1, bf1, g2, bb2, Wf2, bf2)


# speedup vs baseline: 3.1394x; 1.1383x over previous
import functools

import jax
import jax.numpy as jnp
from jax import lax
from jax.experimental import pallas as pl
from jax.experimental.pallas import tpu as pltpu
from jax.experimental.pallas import tpu_sc as plsc

N = 10000
E = 160000
D = 256
DE = 16
H = 16
G = 64
C = 10

NT = 16
K2_CHUNK = 80
K4_CHUNK = 40
ROWS_PER_TILE = 624


def _k1a_body(ea_ref, we1_ref, be1_ref, e1t_ref):
    e1t_ref[...] = jnp.dot(ea_ref[...], we1_ref[...],
                           preferred_element_type=jnp.float32) + be1_ref[0]


def _edge_mlp1(edge_attr, We1, be1):
    BE = 4000
    nb = E // BE
    return pl.pallas_call(
        _k1a_body,
        grid=(2, nb),
        in_specs=[
            pl.BlockSpec((BE, DE), lambda h, i: (i, 0)),
            pl.BlockSpec((DE, 128), lambda h, i: (0, h)),
            pl.BlockSpec((1, 1, 128), lambda h, i: (h, 0, 0)),
        ],
        out_specs=pl.BlockSpec((BE, 128), lambda h, i: (h * nb + i, 0)),
        out_shape=jax.ShapeDtypeStruct((2 * E, 128), jnp.float32),
    )(edge_attr, We1, be1.reshape(2, 1, 128))


def _k1b_body(ea_ref, we2_ref, be2_ref, e2_ref):
    e2_ref[...] = jnp.dot(ea_ref[...], we2_ref[...],
                          preferred_element_type=jnp.float32) + be2_ref[...]


def _edge_mlp2(edge_attr, We2, be2):
    BE = 8000
    nb = E // BE
    return pl.pallas_call(
        _k1b_body,
        grid=(nb,),
        in_specs=[
            pl.BlockSpec((BE, DE), lambda i: (i, 0)),
            pl.BlockSpec((DE, H), lambda i: (0, 0)),
            pl.BlockSpec((1, H), lambda i: (0, 0)),
        ],
        out_specs=pl.BlockSpec((BE, H), lambda i: (i, 0)),
        out_shape=jax.ShapeDtypeStruct((E, H), jnp.float32),
    )(edge_attr, We2, be2.reshape(1, H))


NB = 2


def _k2_body(xcat, e1t, srccat, dst, out, acc, sidx, didx, gv, eb,
             sem_in, sem_g, sem_sc):
    c = lax.axis_index("c")
    s = lax.axis_index("s")
    per_tile = E // NT
    n_chunks = per_tile // K2_CHUNK

    def _zrow(r, _):
        for j in range(128 // 16):
            gv[0, r, pl.ds(j * 16, 16)] = jnp.zeros((16,), jnp.float32)
        return 0

    lax.fori_loop(0, K2_CHUNK, _zrow, 0)
    row0 = s * ROWS_PER_TILE
    for k in range(7):
        pltpu.sync_copy(gv.at[0], acc.at[pl.ds(row0 + k * K2_CHUNK, K2_CHUNK)])
    pltpu.sync_copy(gv.at[0, pl.ds(0, 64)], acc.at[pl.ds(row0 + 560, 64)])

    @pl.when(s == NT - 1)
    def _():
        pltpu.sync_copy(gv.at[0, pl.ds(0, 16)], acc.at[pl.ds(N - 16, 16)])

    plsc.subcore_barrier()

    def _in_copies(i, j, b):
        base = s * per_tile + i * K2_CHUNK
        pltpu.async_copy(srccat.at[pl.ds(c * E + base, K2_CHUNK)],
                         sidx.at[j], sem_in.at[j])
        pltpu.async_copy(dst.at[pl.ds(base, K2_CHUNK)], didx.at[j],
                         sem_in.at[j])
        pltpu.async_copy(e1t.at[pl.ds(c * E + base, K2_CHUNK)], eb.at[b],
                         sem_in.at[j])

    def _wait_in(i, j, b):
        base = s * per_tile + i * K2_CHUNK
        pltpu.make_async_copy(srccat.at[pl.ds(c * E + base, K2_CHUNK)],
                              sidx.at[j], sem_in.at[j]).wait()
        pltpu.make_async_copy(dst.at[pl.ds(base, K2_CHUNK)], didx.at[j],
                              sem_in.at[j]).wait()
        pltpu.make_async_copy(e1t.at[pl.ds(c * E + base, K2_CHUNK)],
                              eb.at[b], sem_in.at[j]).wait()

    def _gather(j, b):
        pltpu.async_copy(xcat.at[sidx.at[j]], gv.at[b], sem_g.at[b])

    def _wait_gather(j, b):
        pltpu.make_async_copy(xcat.at[sidx.at[j]], gv.at[b],
                              sem_g.at[b]).wait()

    def _compute(b):
        def _row(r, _):
            for rr in range(2):
                for j in range(128 // 16):
                    sl = pl.ds(j * 16, 16)
                    gv[b, 2 * r + rr, sl] = jnp.maximum(
                        gv[b, 2 * r + rr, sl] + eb[b, 2 * r + rr, sl], 0.0)
            return 0

        lax.fori_loop(0, K2_CHUNK // 2, _row, 0)

    def _scatter(j, b):
        pltpu.async_copy(gv.at[b], acc.at[didx.at[j]], sem_sc.at[b],
                         add=True)

    def _wait_sc(j, b):
        pltpu.make_async_copy(gv.at[b], acc.at[didx.at[j]],
                              sem_sc.at[b]).wait()

    J = lambda i: i % 4
    B = lambda i: i % 2

    def _head(i, first=False):
        _wait_in(i + 1, J(i + 1), B(i + 1))
        if not first:
            _wait_sc(J(i - 1), B(i - 1))
        _gather(J(i + 1), B(i + 1))

    def _tail(i, do_copies=True):
        _wait_gather(J(i), B(i))
        _compute(B(i))
        _scatter(J(i), B(i))
        if do_copies:
            _in_copies(i + 2, J(i + 2), B(i + 2))

    _in_copies(0, 0, 0)
    _in_copies(1, 1, 1)
    _wait_in(0, 0, 0)
    _gather(0, 0)
    _head(0, first=True)
    _tail(0)

    def _outer(o, _):
        for k in range(4):
            i = 1 + o * 4 + k
            _head(i)
            _tail(i)
        return 0

    lax.fori_loop(0, 30, _outer, 0)
    for i in (121, 122):
        _head(i)
        _tail(i)
    _head(123)
    _tail(123, do_copies=False)
    _wait_sc(J(123), B(123))
    _wait_gather(J(124), B(124))
    _compute(B(124))
    _scatter(J(124), B(124))
    _wait_sc(J(124), B(124))

    plsc.subcore_barrier()
    pltpu.sync_copy(acc.at[pl.ds(row0, ROWS_PER_TILE)],
                    out.at[pl.ds(c * N + row0, ROWS_PER_TILE)])

    @pl.when(s == NT - 1)
    def _():
        pltpu.sync_copy(acc.at[pl.ds(N - 16, 16)],
                        out.at[pl.ds(c * N + N - 16, 16)])


def _layer1_aggregate(xcat, e1t, srccat, dst):
    mesh = plsc.VectorSubcoreMesh(core_axis_name="c", subcore_axis_name="s")
    return pl.kernel(
        _k2_body,
        mesh=mesh,
        out_type=jax.ShapeDtypeStruct((2 * N, 128), jnp.float32),
        scratch_types=[
            pltpu.VMEM_SHARED((N, 128), jnp.float32),
            pltpu.VMEM((4, K2_CHUNK), jnp.int32),
            pltpu.VMEM((4, K2_CHUNK), jnp.int32),
            pltpu.VMEM((2, K2_CHUNK, 128), jnp.float32),
            pltpu.VMEM((2, K2_CHUNK, 128), jnp.float32),
            pltpu.SemaphoreType.DMA((4,)),
            pltpu.SemaphoreType.DMA((2,)),
            pltpu.SemaphoreType.DMA((2,)),
        ],
    )(xcat, e1t, srccat, dst)


def _k3_body(x_ref, a0_ref, a1_ref, w11_ref, b11_ref, w12_ref, b12_ref,
             g1_ref, bb1_ref, out_ref):
    h = x_ref[...] + jnp.concatenate([a0_ref[...], a1_ref[...]], axis=-1)
    t = jnp.maximum(jnp.dot(h, w11_ref[...],
                            preferred_element_type=jnp.float32) + b11_ref[...], 0.0)
    t = jnp.dot(t, w12_ref[...], preferred_element_type=jnp.float32) + b12_ref[...]
    t = jnp.maximum(t, 0.0)
    mu = jnp.mean(t, axis=-1, keepdims=True)
    var = jnp.mean((t - mu) ** 2, axis=-1, keepdims=True)
    t = (t - mu) / jnp.sqrt(var + 1e-5) * g1_ref[...] + bb1_ref[...]
    out_ref[...] = jnp.concatenate(
        [t, jnp.zeros((t.shape[0], 128 - H), jnp.float32)], axis=-1)


def _node_mlp1(x, a0, a1, W11, b11, W12, b12, g1, bb1):
    BN = 1000
    nb = N // BN
    return pl.pallas_call(
        _k3_body,
        grid=(nb,),
        in_specs=[
            pl.BlockSpec((BN, D), lambda i: (i, 0)),
            pl.BlockSpec((BN, 128), lambda i: (i, 0)),
            pl.BlockSpec((BN, 128), lambda i: (i, 0)),
            pl.BlockSpec((D, H), lambda i: (0, 0)),
            pl.BlockSpec((1, H), lambda i: (0, 0)),
            pl.BlockSpec((H, H), lambda i: (0, 0)),
            pl.BlockSpec((1, H), lambda i: (0, 0)),
            pl.BlockSpec((1, H), lambda i: (0, 0)),
            pl.BlockSpec((1, H), lambda i: (0, 0)),
        ],
        out_specs=pl.BlockSpec((BN, 128), lambda i: (i, 0)),
        out_shape=jax.ShapeDtypeStruct((N, 128), jnp.float32),
    )(x, a0, a1, W11, b11.reshape(1, H), W12, b12.reshape(1, H),
      g1.reshape(1, H), bb1.reshape(1, H))


def _k4_body(h1, e2, src, dst, out, acc, sidx, didx, gv, ev,
             sem_in, sem_g, sem_sc):
    c = lax.axis_index("c")
    s = lax.axis_index("s")
    w = s * 2 + c
    per_w = E // 32
    n_chunks = per_w // K4_CHUNK

    def _zrow(r, _):
        for j in range(128 // 16):
            gv[0, r, pl.ds(j * 16, 16)] = jnp.zeros((16,), jnp.float32)
        return 0

    lax.fori_loop(0, K4_CHUNK, _zrow, 0)
    row0 = s * ROWS_PER_TILE
    for k in range(15):
        pltpu.sync_copy(gv.at[0], acc.at[pl.ds(row0 + k * K4_CHUNK, K4_CHUNK)])
    pltpu.sync_copy(gv.at[0, pl.ds(0, 24)], acc.at[pl.ds(row0 + 600, 24)])

    @pl.when(s == NT - 1)
    def _():
        pltpu.sync_copy(gv.at[0, pl.ds(0, 16)], acc.at[pl.ds(N - 16, 16)])

    plsc.subcore_barrier()

    def _in_copies(i, j, b):
        base = w * per_w + i * K4_CHUNK
        pltpu.async_copy(src.at[pl.ds(base, K4_CHUNK)], sidx.at[j],
                         sem_in.at[j])
        pltpu.async_copy(dst.at[pl.ds(base, K4_CHUNK)], didx.at[j],
                         sem_in.at[j])
        pltpu.async_copy(e2.at[pl.ds(base, K4_CHUNK)], ev.at[b],
                         sem_in.at[j])

    def _wait_in(i, j, b):
        base = w * per_w + i * K4_CHUNK
        pltpu.make_async_copy(src.at[pl.ds(base, K4_CHUNK)], sidx.at[j],
                              sem_in.at[j]).wait()
        pltpu.make_async_copy(dst.at[pl.ds(base, K4_CHUNK)], didx.at[j],
                              sem_in.at[j]).wait()
        pltpu.make_async_copy(e2.at[pl.ds(base, K4_CHUNK)], ev.at[b],
                              sem_in.at[j]).wait()

    def _gather(j, b):
        pltpu.async_copy(h1.at[sidx.at[j]], gv.at[b], sem_g.at[b])

    def _wait_gather(j, b):
        pltpu.make_async_copy(h1.at[sidx.at[j]], gv.at[b],
                              sem_g.at[b]).wait()

    def _compute(b):
        def _row(r, _):
            for rr in range(4):
                sl = pl.ds(0, 16)
                gv[b, 4 * r + rr, sl] = jnp.maximum(
                    gv[b, 4 * r + rr, sl] + ev[b, 4 * r + rr, :], 0.0)
            return 0

        lax.fori_loop(0, K4_CHUNK // 4, _row, 0)

    def _scatter(j, b):
        pltpu.async_copy(gv.at[b], acc.at[didx.at[j]], sem_sc.at[b],
                         add=True)

    def _wait_sc(j, b):
        pltpu.make_async_copy(gv.at[b], acc.at[didx.at[j]],
                              sem_sc.at[b]).wait()

    J = lambda i: i % 4
    B = lambda i: i % 2

    def _head(i, first=False):
        _wait_in(i + 1, J(i + 1), B(i + 1))
        if not first:
            _wait_sc(J(i - 1), B(i - 1))
        _gather(J(i + 1), B(i + 1))

    def _tail(i, do_copies=True):
        _wait_gather(J(i), B(i))
        _compute(B(i))
        _scatter(J(i), B(i))
        if do_copies:
            _in_copies(i + 2, J(i + 2), B(i + 2))

    _in_copies(0, 0, 0)
    _in_copies(1, 1, 1)
    _wait_in(0, 0, 0)
    _gather(0, 0)
    _head(0, first=True)
    _tail(0)

    def _outer(o, _):
        for k in range(4):
            i = 1 + o * 4 + k
            _head(i)
            _tail(i)
        return 0

    lax.fori_loop(0, 30, _outer, 0)
    for i in (121, 122):
        _head(i)
        _tail(i)
    _head(123)
    _tail(123, do_copies=False)
    _wait_sc(J(123), B(123))
    _wait_gather(J(124), B(124))
    _compute(B(124))
    _scatter(J(124), B(124))
    _wait_sc(J(124), B(124))

    plsc.subcore_barrier()
    pltpu.sync_copy(acc.at[pl.ds(row0, ROWS_PER_TILE)],
                    out.at[pl.ds(c * N + row0, ROWS_PER_TILE)])

    @pl.when(s == NT - 1)
    def _():
        pltpu.sync_copy(acc.at[pl.ds(N - 16, 16)],
                        out.at[pl.ds(c * N + N - 16, 16)])


def _layer2_aggregate(h1, e2, src, dst):
    mesh = plsc.VectorSubcoreMesh(core_axis_name="c", subcore_axis_name="s")
    return pl.kernel(
        _k4_body,
        mesh=mesh,
        out_type=jax.ShapeDtypeStruct((2 * N, 128), jnp.float32),
        scratch_types=[
            pltpu.VMEM_SHARED((N, 128), jnp.float32),
            pltpu.VMEM((4, K4_CHUNK), jnp.int32),
            pltpu.VMEM((4, K4_CHUNK), jnp.int32),
            pltpu.VMEM((2, K4_CHUNK, 128), jnp.float32),
            pltpu.VMEM((2, K4_CHUNK, H), jnp.float32),
            pltpu.SemaphoreType.DMA((4,)),
            pltpu.SemaphoreType.DMA((2,)),
            pltpu.SemaphoreType.DMA((2,)),
        ],
    )(h1, e2, src, dst)


def _k5_body(h1_ref, p0_ref, p1_ref, b_ref, w21_ref, b21_ref, w22_ref,
             b22_ref, wf1_ref, bf1_ref, g2_ref, bb2_ref, wf2_ref, bf2_ref,
             out_ref, pooled):
    i = pl.program_id(0)
    nb = pl.num_programs(0)
    hh = (h1_ref[:, :H] + p0_ref[:, :H] + p1_ref[:, :H])
    t = jnp.maximum(jnp.dot(hh, w21_ref[...],
                            preferred_element_type=jnp.float32) + b21_ref[...], 0.0)
    t = jnp.dot(t, w22_ref[...], preferred_element_type=jnp.float32) + b22_ref[...]
    h2 = jnp.maximum(t, 0.0)
    bid = b_ref[...]
    mask = bid == lax.broadcasted_iota(jnp.int32, (1, G), 1)
    rows = []
    for f in range(H):
        wf = jnp.where(mask, h2[:, f:f + 1], -jnp.inf)
        rows.append(jnp.max(wf, axis=0, keepdims=True))
    bmax = jnp.concatenate(rows, axis=0)

    @pl.when(i == 0)
    def _():
        pooled[...] = jnp.full((H, G), -jnp.inf, jnp.float32)

    pooled[...] = jnp.maximum(pooled[...], bmax)

    @pl.when(i == nb - 1)
    def _():
        y = lax.dot_general(pooled[...], wf1_ref[...],
                            (((0,), (0,)), ((), ())),
                            preferred_element_type=jnp.float32) + bf1_ref[...]
        mu = jnp.mean(y, axis=-1, keepdims=True)
        var = jnp.mean((y - mu) ** 2, axis=-1, keepdims=True)
        y = (y - mu) / jnp.sqrt(var + 1e-5) * g2_ref[...] + bb2_ref[...]
        y = jnp.maximum(y, 0.0)
        y = jnp.dot(y, wf2_ref[...],
                    preferred_element_type=jnp.float32) + bf2_ref[...]
        m = jnp.max(y, axis=-1, keepdims=True)
        z = y - m
        out_ref[...] = z - jnp.log(jnp.sum(jnp.exp(z), axis=-1, keepdims=True))


def _pool_head(h1, p0, p1, batch, W21, b21, W22, b22, Wf1, bf1, g2, bb2,
               Wf2, bf2):
    BN = 1000
    nb = N // BN
    full = lambda i: (0, 0)
    return pl.pallas_call(
        _k5_body,
        grid=(nb,),
        in_specs=[
            pl.BlockSpec((BN, 128), lambda i: (i, 0)),
            pl.BlockSpec((BN, 128), lambda i: (i, 0)),
            pl.BlockSpec((BN, 128), lambda i: (i, 0)),
            pl.BlockSpec((BN, 1), lambda i: (i, 0)),
            pl.BlockSpec((H, H), full),
            pl.BlockSpec((1, H), full),
            pl.BlockSpec((H, H), full),
            pl.BlockSpec((1, H), full),
            pl.BlockSpec((H, 32), full),
            pl.BlockSpec((1, 32), full),
            pl.BlockSpec((1, 32), full),
            pl.BlockSpec((1, 32), full),
            pl.BlockSpec((32, C), full),
            pl.BlockSpec((1, C), full),
        ],
        out_specs=pl.BlockSpec((G, C), full),
        out_shape=jax.ShapeDtypeStruct((G, C), jnp.float32),
        scratch_shapes=[pltpu.VMEM((H, G), jnp.float32)],
    )(h1, p0, p1, batch.reshape(N, 1), W21, b21.reshape(1, H), W22,
      b22.reshape(1, H), Wf1, bf1.reshape(1, 32), g2.reshape(1, 32),
      bb2.reshape(1, 32), Wf2, bf2.reshape(1, C))


def kernel(x, edge_index, edge_attr, batch, We1, be1, W11, b11, W12, b12,
           We2, be2, W21, b21, W22, b22, g1, bb1, Wf1, bf1, g2, bb2, Wf2,
           bf2):
    src = edge_index[0]
    dst = edge_index[1]
    srccat = jnp.concatenate([src, src + N])
    e1t = _edge_mlp1(edge_attr, We1, be1)
    e2 = _edge_mlp2(edge_attr, We2, be2)
    xcat = jnp.concatenate([x[:, :128], x[:, 128:]], axis=0)
    aggr = _layer1_aggregate(xcat, e1t, srccat, dst)
    h1 = _node_mlp1(x, aggr[:N], aggr[N:], W11, b11, W12, b12, g1, bb1)
    pp = _layer2_aggregate(h1, e2, src, dst)
    return _pool_head(h1, pp[:N], pp[N:], batch, W21, b21, W22, b22,
                      Wf1, bf1, g2, bb2, Wf2, bf2)
